# baseline (speedup 1.0000x reference)
import functools

import jax
import jax.numpy as jnp
import numpy as np
from jax import lax
from jax.experimental import pallas as pl
from jax.experimental.pallas import tpu as pltpu
from jax.experimental.pallas import tpu_sc as plsc

E = 3_200_000
F = 3
VOCAB = 8
EMB = 16

NC, NS = 2, 16
NW = NC * NS
CHUNK = 3200
N_CHUNKS = E // CHUNK
MAX_ITERS = -(-N_CHUNKS // NW)
_GCHUNKS = [(k * 128, 128) for k in range(CHUNK // 128)]
N_GROUPS = E // 128
GROUPS_PER_CHUNK = CHUNK // 128

_PACK_R = E // 128
_PACK_BLK = 1000
_W = np.zeros((3 * 128, 128), np.float32)
_W[3 * np.arange(128) + 0, np.arange(128)] = 64.0
_W[3 * np.arange(128) + 1, np.arange(128)] = 8.0
_W[3 * np.arange(128) + 2, np.arange(128)] = 1.0


def _combine_body(t0_ref, t1_ref, t2_ref, c_ref):
    t0 = t0_ref[...]
    t1 = t1_ref[...]
    t2 = t2_ref[...]
    x = t0[:, None, None, :] + t1[None, :, None, :] + t2[None, None, :, :]
    c_ref[...] = x.reshape(VOCAB ** 3, EMB)


def _build_combined(T0, T1, T2):
    return pl.pallas_call(
        _combine_body,
        out_shape=jax.ShapeDtypeStruct((VOCAB ** 3, EMB), jnp.float32),
    )(T0, T1, T2)


def _pack_body(attr_ref, w_ref, idx_ref):
    x = attr_ref[...].astype(jnp.float32)
    y = jax.lax.dot(x, w_ref[...], preferred_element_type=jnp.float32)
    idx_ref[...] = y.astype(jnp.int32)


def _pack_indices(edge_attr):
    flat = edge_attr.reshape(_PACK_R, 3 * 128)
    idx = pl.pallas_call(
        _pack_body,
        grid=(_PACK_R // _PACK_BLK,),
        in_specs=[
            pl.BlockSpec((_PACK_BLK, 3 * 128), lambda i: (i, 0)),
            pl.BlockSpec((3 * 128, 128), lambda i: (0, 0)),
        ],
        out_specs=pl.BlockSpec((_PACK_BLK, 128), lambda i: (i, 0)),
        out_shape=jax.ShapeDtypeStruct((_PACK_R, 128), jnp.int32),
    )(flat, jnp.asarray(_W))
    return idx.reshape(E)


@functools.partial(
    pl.kernel,
    out_type=jax.ShapeDtypeStruct((E, EMB), jnp.float32),
    mesh=plsc.VectorSubcoreMesh(core_axis_name="c", subcore_axis_name="s"),
    compiler_params=pltpu.CompilerParams(use_tc_tiling_on_sc=False),
    scratch_types=[
        pltpu.VMEM((CHUNK,), jnp.int32),
        pltpu.VMEM((CHUNK, EMB), jnp.float32),
        pltpu.SemaphoreType.DMA,
    ],
)
def _sc_encode(idx_hbm, c_hbm, out_hbm, idx_v, rows_v, gsem):
    wid = lax.axis_index("s") * NC + lax.axis_index("c")

    def outer(i, carry):
        cid = wid + i * NW

        @pl.when(cid < N_CHUNKS)
        def _():
            pltpu.sync_copy(idx_hbm.at[pl.ds(cid * CHUNK, CHUNK)], idx_v)
            handles = [
                pltpu.async_copy(
                    c_hbm.at[idx_v.at[pl.ds(off, sz)]],
                    rows_v.at[pl.ds(off, sz)],
                    gsem,
                )
                for off, sz in _GCHUNKS
            ]
            for h in handles:
                h.wait()
            pltpu.sync_copy(
                rows_v,
                out_hbm.at[pl.ds(cid * CHUNK, CHUNK)],
            )

        return carry

    lax.fori_loop(0, MAX_ITERS, outer, 0)


def kernel(edge_attr, T0, T1, T2):
    c = _build_combined(T0, T1, T2)
    idx = _pack_indices(edge_attr)
    return _sc_encode(idx, c)


# speedup vs baseline: 2.9818x; 2.9818x over previous
import functools

import jax
import jax.numpy as jnp
import numpy as np
from jax import lax
from jax.experimental import pallas as pl
from jax.experimental.pallas import tpu as pltpu
from jax.experimental.pallas import tpu_sc as plsc

E = 3_200_000
F = 3
VOCAB = 8
EMB = 16

NC, NS = 2, 16
NW = NC * NS
CHUNK = 3200
N_CHUNKS = E // CHUNK
MAX_ITERS = -(-N_CHUNKS // NW)
_GCHUNKS = [(k * 128, 128) for k in range(CHUNK // 128)]
N_GROUPS = E // 128
GROUPS_PER_CHUNK = CHUNK // 128

_PACK_EDGES = 5120
_PACK_OUT_R = _PACK_EDGES // 128


def _combine_body(t0_ref, t1_ref, t2_ref, c_ref):
    t0 = t0_ref[...]
    t1 = t1_ref[...]
    t2 = t2_ref[...]
    x = t0[:, None, None, :] + t1[None, :, None, :] + t2[None, None, :, :]
    c_ref[...] = x.reshape(VOCAB ** 3, EMB)


def _build_combined(T0, T1, T2):
    return pl.pallas_call(
        _combine_body,
        out_shape=jax.ShapeDtypeStruct((VOCAB ** 3, EMB), jnp.float32),
    )(T0, T1, T2)


def _pack_body(attr_ref, idx_ref):
    x = attr_ref[...]
    packed = x[:, 0] * 64 + x[:, 1] * 8 + x[:, 2]
    idx_ref[...] = packed.reshape(_PACK_OUT_R, 128)


def _pack_indices(edge_attr):
    idx = pl.pallas_call(
        _pack_body,
        grid=(E // _PACK_EDGES,),
        in_specs=[pl.BlockSpec((_PACK_EDGES, F), lambda i: (i, 0))],
        out_specs=pl.BlockSpec((_PACK_OUT_R, 128), lambda i: (i, 0)),
        out_shape=jax.ShapeDtypeStruct((E // 128, 128), jnp.int32),
    )(edge_attr)
    return idx.reshape(E)


@functools.partial(
    pl.kernel,
    out_type=jax.ShapeDtypeStruct((E, EMB), jnp.float32),
    mesh=plsc.VectorSubcoreMesh(core_axis_name="c", subcore_axis_name="s"),
    compiler_params=pltpu.CompilerParams(use_tc_tiling_on_sc=False),
    scratch_types=[
        pltpu.VMEM((CHUNK,), jnp.int32),
        pltpu.VMEM((CHUNK, EMB), jnp.float32),
        pltpu.SemaphoreType.DMA,
    ],
)
def _sc_encode(idx_hbm, c_hbm, out_hbm, idx_v, rows_v, gsem):
    wid = lax.axis_index("s") * NC + lax.axis_index("c")

    def outer(i, carry):
        cid = wid + i * NW

        @pl.when(cid < N_CHUNKS)
        def _():
            pltpu.sync_copy(idx_hbm.at[pl.ds(cid * CHUNK, CHUNK)], idx_v)
            handles = [
                pltpu.async_copy(
                    c_hbm.at[idx_v.at[pl.ds(off, sz)]],
                    rows_v.at[pl.ds(off, sz)],
                    gsem,
                )
                for off, sz in _GCHUNKS
            ]
            for h in handles:
                h.wait()
            pltpu.sync_copy(
                rows_v,
                out_hbm.at[pl.ds(cid * CHUNK, CHUNK)],
            )

        return carry

    lax.fori_loop(0, MAX_ITERS, outer, 0)


def kernel(edge_attr, T0, T1, T2):
    c = _build_combined(T0, T1, T2)
    idx = _pack_indices(edge_attr)
    return _sc_encode(idx, c)


# speedup vs baseline: 5.9106x; 1.9822x over previous
import functools

import jax
import jax.numpy as jnp
import numpy as np
from jax import lax
from jax.experimental import pallas as pl
from jax.experimental.pallas import tpu as pltpu
from jax.experimental.pallas import tpu_sc as plsc

E = 3_200_000
F = 3
VOCAB = 8
EMB = 16

NC, NS = 2, 16
NW = NC * NS
CHUNK = 3200
N_CHUNKS = E // CHUNK
MAX_ITERS = -(-N_CHUNKS // NW)
_GCHUNKS = [(k * 128, 128) for k in range(CHUNK // 128)]
N_GROUPS = E // 128
GROUPS_PER_CHUNK = CHUNK // 128

_PACK_EDGES = 128000
_PACK_OUT_R = _PACK_EDGES // 128


def _combine_body(t0_ref, t1_ref, t2_ref, c_ref):
    t0 = t0_ref[...]
    t1 = t1_ref[...]
    t2 = t2_ref[...]
    x = t0[:, None, None, :] + t1[None, :, None, :] + t2[None, None, :, :]
    c_ref[...] = x.reshape(VOCAB ** 3, EMB)


def _build_combined(T0, T1, T2):
    return pl.pallas_call(
        _combine_body,
        out_shape=jax.ShapeDtypeStruct((VOCAB ** 3, EMB), jnp.float32),
    )(T0, T1, T2)


def _pack_body(attr_ref, idx_ref):
    x = attr_ref[...]
    packed = x[0] * 64 + x[1] * 8 + x[2]
    idx_ref[...] = packed.reshape(_PACK_OUT_R, 128)


def _pack_indices(edge_attr):
    attr_t = edge_attr.T
    idx = pl.pallas_call(
        _pack_body,
        grid=(E // _PACK_EDGES,),
        in_specs=[pl.BlockSpec((F, _PACK_EDGES), lambda i: (0, i))],
        out_specs=pl.BlockSpec((_PACK_OUT_R, 128), lambda i: (i, 0)),
        out_shape=jax.ShapeDtypeStruct((E // 128, 128), jnp.int32),
    )(attr_t)
    return idx.reshape(E)


@functools.partial(
    pl.kernel,
    out_type=jax.ShapeDtypeStruct((E, EMB), jnp.float32),
    mesh=plsc.VectorSubcoreMesh(core_axis_name="c", subcore_axis_name="s"),
    compiler_params=pltpu.CompilerParams(use_tc_tiling_on_sc=False),
    scratch_types=[
        pltpu.VMEM((CHUNK,), jnp.int32),
        pltpu.VMEM((CHUNK, EMB), jnp.float32),
        pltpu.SemaphoreType.DMA,
    ],
)
def _sc_encode(idx_hbm, c_hbm, out_hbm, idx_v, rows_v, gsem):
    wid = lax.axis_index("s") * NC + lax.axis_index("c")

    def outer(i, carry):
        cid = wid + i * NW

        @pl.when(cid < N_CHUNKS)
        def _():
            pltpu.sync_copy(idx_hbm.at[pl.ds(cid * CHUNK, CHUNK)], idx_v)
            handles = [
                pltpu.async_copy(
                    c_hbm.at[idx_v.at[pl.ds(off, sz)]],
                    rows_v.at[pl.ds(off, sz)],
                    gsem,
                )
                for off, sz in _GCHUNKS
            ]
            for h in handles:
                h.wait()
            pltpu.sync_copy(
                rows_v,
                out_hbm.at[pl.ds(cid * CHUNK, CHUNK)],
            )

        return carry

    lax.fori_loop(0, MAX_ITERS, outer, 0)


def kernel(edge_attr, T0, T1, T2):
    c = _build_combined(T0, T1, T2)
    idx = _pack_indices(edge_attr)
    return _sc_encode(idx, c)


# speedup vs baseline: 5.9561x; 1.0077x over previous
import functools

import jax
import jax.numpy as jnp
import numpy as np
from jax import lax
from jax.experimental import pallas as pl
from jax.experimental.pallas import tpu as pltpu
from jax.experimental.pallas import tpu_sc as plsc

E = 3_200_000
F = 3
VOCAB = 8
EMB = 16

NC, NS = 2, 16
NW = NC * NS
CHUNK = 3200
N_CHUNKS = E // CHUNK
MAX_ITERS = -(-N_CHUNKS // NW)
_GCHUNKS = [(k * 128, 128) for k in range(CHUNK // 128)]
N_GROUPS = E // 128
GROUPS_PER_CHUNK = CHUNK // 128

_PACK_EDGES = 128000
_PACK_OUT_R = _PACK_EDGES // 128


def _combine_body(t0_ref, t1_ref, t2_ref, c_ref):
    t0 = t0_ref[...]
    t1 = t1_ref[...]
    t2 = t2_ref[...]
    x = t0[:, None, None, :] + t1[None, :, None, :] + t2[None, None, :, :]
    c_ref[...] = x.reshape(VOCAB ** 3, EMB)


def _build_combined(T0, T1, T2):
    return pl.pallas_call(
        _combine_body,
        out_shape=jax.ShapeDtypeStruct((VOCAB ** 3, EMB), jnp.float32),
    )(T0, T1, T2)


def _pack_body(attr_ref, idx_ref):
    x = attr_ref[...]
    packed = x[0] * 64 + x[1] * 8 + x[2]
    idx_ref[...] = packed.reshape(_PACK_OUT_R, 128)


def _pack_indices(edge_attr):
    attr_t = edge_attr.T
    idx = pl.pallas_call(
        _pack_body,
        grid=(E // _PACK_EDGES,),
        in_specs=[pl.BlockSpec((F, _PACK_EDGES), lambda i: (0, i))],
        out_specs=pl.BlockSpec((_PACK_OUT_R, 128), lambda i: (i, 0)),
        out_shape=jax.ShapeDtypeStruct((E // 128, 128), jnp.int32),
    )(attr_t)
    return idx.reshape(E)


@functools.partial(
    pl.kernel,
    out_type=jax.ShapeDtypeStruct((E, EMB), jnp.float32),
    mesh=plsc.VectorSubcoreMesh(core_axis_name="c", subcore_axis_name="s"),
    compiler_params=pltpu.CompilerParams(use_tc_tiling_on_sc=False),
    scratch_types=[
        pltpu.VMEM((2, CHUNK), jnp.int32),
        pltpu.VMEM((2, CHUNK, EMB), jnp.float32),
        pltpu.SemaphoreType.DMA,
        pltpu.SemaphoreType.DMA,
        pltpu.SemaphoreType.DMA,
    ],
)
def _sc_encode(idx_hbm, c_hbm, out_hbm, idx_v, rows_v, isem, gsem, osem):
    wid = lax.axis_index("s") * NC + lax.axis_index("c")
    n_valid = (N_CHUNKS - wid + NW - 1) // NW

    def idx_slice(i):
        return idx_hbm.at[pl.ds((wid + i * NW) * CHUNK, CHUNK)]

    def out_slice(i):
        return out_hbm.at[pl.ds((wid + i * NW) * CHUNK, CHUNK)]

    @pl.when(n_valid > 0)
    def _():
        pltpu.async_copy(idx_slice(0), idx_v.at[0], isem)

    def body(i, carry):
        p = lax.rem(i, 2)

        @pl.when(i >= 2)
        def _():
            pltpu.make_async_copy(rows_v.at[p], out_slice(i), osem).wait()

        pltpu.make_async_copy(idx_slice(i), idx_v.at[p], isem).wait()

        @pl.when(i + 1 < n_valid)
        def _():
            pltpu.async_copy(idx_slice(i + 1), idx_v.at[1 - p], isem)

        handles = [
            pltpu.async_copy(
                c_hbm.at[idx_v.at[p].at[pl.ds(off, sz)]],
                rows_v.at[p].at[pl.ds(off, sz)],
                gsem,
            )
            for off, sz in _GCHUNKS
        ]
        for h in handles:
            h.wait()
        pltpu.async_copy(rows_v.at[p], out_slice(i), osem)
        return carry

    lax.fori_loop(0, n_valid, body, 0)

    @pl.when(n_valid >= 1)
    def _():
        pltpu.make_async_copy(rows_v.at[0], out_slice(0), osem).wait()

    @pl.when(n_valid >= 2)
    def _():
        pltpu.make_async_copy(rows_v.at[1], out_slice(0), osem).wait()


def kernel(edge_attr, T0, T1, T2):
    c = _build_combined(T0, T1, T2)
    idx = _pack_indices(edge_attr)
    return _sc_encode(idx, c)


# speedup vs baseline: 5.9570x; 1.0002x over previous
import functools

import jax
import jax.numpy as jnp
import numpy as np
from jax import lax
from jax.experimental import pallas as pl
from jax.experimental.pallas import tpu as pltpu
from jax.experimental.pallas import tpu_sc as plsc

E = 3_200_000
F = 3
VOCAB = 8
EMB = 16

NC, NS = 2, 16
NW = NC * NS
CHUNK = 3200
N_CHUNKS = E // CHUNK
MAX_ITERS = -(-N_CHUNKS // NW)
_GCHUNKS = [(0, CHUNK)]
N_GROUPS = E // 128
GROUPS_PER_CHUNK = CHUNK // 128

_PACK_EDGES = 128000
_PACK_OUT_R = _PACK_EDGES // 128


def _combine_body(t0_ref, t1_ref, t2_ref, c_ref):
    t0 = t0_ref[...]
    t1 = t1_ref[...]
    t2 = t2_ref[...]
    x = t0[:, None, None, :] + t1[None, :, None, :] + t2[None, None, :, :]
    c_ref[...] = x.reshape(VOCAB ** 3, EMB)


def _build_combined(T0, T1, T2):
    return pl.pallas_call(
        _combine_body,
        out_shape=jax.ShapeDtypeStruct((VOCAB ** 3, EMB), jnp.float32),
    )(T0, T1, T2)


def _pack_body(attr_ref, idx_ref):
    x = attr_ref[...]
    packed = x[0] * 64 + x[1] * 8 + x[2]
    idx_ref[...] = packed.reshape(_PACK_OUT_R, 128)


def _pack_indices(edge_attr):
    attr_t = edge_attr.T
    idx = pl.pallas_call(
        _pack_body,
        grid=(E // _PACK_EDGES,),
        in_specs=[pl.BlockSpec((F, _PACK_EDGES), lambda i: (0, i))],
        out_specs=pl.BlockSpec((_PACK_OUT_R, 128), lambda i: (i, 0)),
        out_shape=jax.ShapeDtypeStruct((E // 128, 128), jnp.int32),
    )(attr_t)
    return idx.reshape(E)


@functools.partial(
    pl.kernel,
    out_type=jax.ShapeDtypeStruct((E, EMB), jnp.float32),
    mesh=plsc.VectorSubcoreMesh(core_axis_name="c", subcore_axis_name="s"),
    compiler_params=pltpu.CompilerParams(use_tc_tiling_on_sc=False),
    scratch_types=[
        pltpu.VMEM((2, CHUNK), jnp.int32),
        pltpu.VMEM((2, CHUNK, EMB), jnp.float32),
        pltpu.SemaphoreType.DMA,
        pltpu.SemaphoreType.DMA,
        pltpu.SemaphoreType.DMA,
    ],
)
def _sc_encode(idx_hbm, c_hbm, out_hbm, idx_v, rows_v, isem, gsem, osem):
    wid = lax.axis_index("s") * NC + lax.axis_index("c")
    n_valid = (N_CHUNKS - wid + NW - 1) // NW

    def idx_slice(i):
        return idx_hbm.at[pl.ds((wid + i * NW) * CHUNK, CHUNK)]

    def out_slice(i):
        return out_hbm.at[pl.ds((wid + i * NW) * CHUNK, CHUNK)]

    @pl.when(n_valid > 0)
    def _():
        pltpu.async_copy(idx_slice(0), idx_v.at[0], isem)

    def body(i, carry):
        p = lax.rem(i, 2)

        @pl.when(i >= 2)
        def _():
            pltpu.make_async_copy(rows_v.at[p], out_slice(i), osem).wait()

        pltpu.make_async_copy(idx_slice(i), idx_v.at[p], isem).wait()

        @pl.when(i + 1 < n_valid)
        def _():
            pltpu.async_copy(idx_slice(i + 1), idx_v.at[1 - p], isem)

        handles = [
            pltpu.async_copy(
                c_hbm.at[idx_v.at[p].at[pl.ds(off, sz)]],
                rows_v.at[p].at[pl.ds(off, sz)],
                gsem,
            )
            for off, sz in _GCHUNKS
        ]
        for h in handles:
            h.wait()
        pltpu.async_copy(rows_v.at[p], out_slice(i), osem)
        return carry

    lax.fori_loop(0, n_valid, body, 0)

    @pl.when(n_valid >= 1)
    def _():
        pltpu.make_async_copy(rows_v.at[0], out_slice(0), osem).wait()

    @pl.when(n_valid >= 2)
    def _():
        pltpu.make_async_copy(rows_v.at[1], out_slice(0), osem).wait()


def kernel(edge_attr, T0, T1, T2):
    c = _build_combined(T0, T1, T2)
    idx = _pack_indices(edge_attr)
    return _sc_encode(idx, c)


# speedup vs baseline: 7.7522x; 1.3014x over previous
import functools

import jax
import jax.numpy as jnp
import numpy as np
from jax import lax
from jax.experimental import pallas as pl
from jax.experimental.pallas import tpu as pltpu
from jax.experimental.pallas import tpu_sc as plsc

E = 3_200_000
F = 3
VOCAB = 8
EMB = 16

NC, NS = 2, 16
NW = NC * NS
CHUNK = 3200
N_CHUNKS = E // CHUNK
MAX_ITERS = -(-N_CHUNKS // NW)
_GCHUNKS = [(0, CHUNK)]
N_GROUPS = E // 128
GROUPS_PER_CHUNK = CHUNK // 128

_PACK_EDGES = 128000
_PACK_OUT_R = _PACK_EDGES // 128


def _combine_body(t0_ref, t1_ref, t2_ref, c_ref):
    t0 = t0_ref[...]
    t1 = t1_ref[...]
    t2 = t2_ref[...]
    x = t0[:, None, None, :] + t1[None, :, None, :] + t2[None, None, :, :]
    c_ref[...] = x.reshape(VOCAB ** 3, EMB)


def _build_combined(T0, T1, T2):
    return pl.pallas_call(
        _combine_body,
        out_shape=jax.ShapeDtypeStruct((VOCAB ** 3, EMB), jnp.float32),
    )(T0, T1, T2)


def _pack_body(attr_ref, idx_ref):
    x = attr_ref[...]
    packed = x[0] * 64 + x[1] * 8 + x[2]
    idx_ref[...] = packed.reshape(_PACK_OUT_R, 128)


def _pack_indices(edge_attr):
    attr_t = edge_attr.T
    idx = pl.pallas_call(
        _pack_body,
        grid=(E // _PACK_EDGES,),
        in_specs=[pl.BlockSpec((F, _PACK_EDGES), lambda i: (0, i))],
        out_specs=pl.BlockSpec((_PACK_OUT_R, 128), lambda i: (i, 0)),
        out_shape=jax.ShapeDtypeStruct((E // 128, 128), jnp.int32),
    )(attr_t)
    return idx.reshape(E)


@functools.partial(
    pl.kernel,
    out_type=jax.ShapeDtypeStruct((E, EMB), jnp.float32),
    mesh=plsc.VectorSubcoreMesh(core_axis_name="c", subcore_axis_name="s"),
    compiler_params=pltpu.CompilerParams(use_tc_tiling_on_sc=False),
    scratch_types=[
        pltpu.VMEM((2, CHUNK), jnp.int32),
        pltpu.VMEM((2, CHUNK, EMB), jnp.float32),
        pltpu.VMEM_SHARED((VOCAB ** 3, EMB), jnp.float32),
        pltpu.SemaphoreType.DMA,
        pltpu.SemaphoreType.DMA,
        pltpu.SemaphoreType.DMA,
    ],
)
def _sc_encode(idx_hbm, c_hbm, out_hbm, idx_v, rows_v, c_sh, isem, gsem, osem):
    wid = lax.axis_index("s") * NC + lax.axis_index("c")
    n_valid = (N_CHUNKS - wid + NW - 1) // NW

    @pl.when(lax.axis_index("s") == 0)
    def _():
        pltpu.sync_copy(c_hbm, c_sh)

    plsc.subcore_barrier()

    def idx_slice(i):
        return idx_hbm.at[pl.ds((wid + i * NW) * CHUNK, CHUNK)]

    def out_slice(i):
        return out_hbm.at[pl.ds((wid + i * NW) * CHUNK, CHUNK)]

    @pl.when(n_valid > 0)
    def _():
        pltpu.async_copy(idx_slice(0), idx_v.at[0], isem)

    def body(i, carry):
        p = lax.rem(i, 2)

        @pl.when(i >= 2)
        def _():
            pltpu.make_async_copy(rows_v.at[p], out_slice(i), osem).wait()

        pltpu.make_async_copy(idx_slice(i), idx_v.at[p], isem).wait()

        @pl.when(i + 1 < n_valid)
        def _():
            pltpu.async_copy(idx_slice(i + 1), idx_v.at[1 - p], isem)

        handles = [
            pltpu.async_copy(
                c_sh.at[idx_v.at[p].at[pl.ds(off, sz)]],
                rows_v.at[p].at[pl.ds(off, sz)],
                gsem,
            )
            for off, sz in _GCHUNKS
        ]
        for h in handles:
            h.wait()
        pltpu.async_copy(rows_v.at[p], out_slice(i), osem)
        return carry

    lax.fori_loop(0, n_valid, body, 0)

    @pl.when(n_valid >= 1)
    def _():
        pltpu.make_async_copy(rows_v.at[0], out_slice(0), osem).wait()

    @pl.when(n_valid >= 2)
    def _():
        pltpu.make_async_copy(rows_v.at[1], out_slice(0), osem).wait()


def kernel(edge_attr, T0, T1, T2):
    c = _build_combined(T0, T1, T2)
    idx = _pack_indices(edge_attr)
    return _sc_encode(idx, c)
